# Initial kernel scaffold; baseline (speedup 1.0000x reference)
import jax
import jax.numpy as jnp
from jax.experimental import pallas as pl


def kernel(inputs):
    raise NotImplementedError("write your pallas kernel here")


# speedup vs baseline: 21.9650x; 21.9650x over previous
import jax
import jax.numpy as jnp
from jax.experimental import pallas as pl
from jax.experimental.pallas import tpu as pltpu

NBINS = 256
H = 512
W = 512
HW = H * W
ROWS = 8
COLS = HW // ROWS
NCHUNK = 4
CH = COLS // NCHUNK


def _otsu_body(g_ref, out_ref):
    gray = g_ref[0]

    gmin = jnp.min(gray)
    gmax = jnp.max(gray)
    scale = NBINS / jnp.maximum(gmax - gmin, 1e-12)
    idx = jnp.clip(((gray - gmin) * scale).astype(jnp.int32), 0, NBINS - 1)
    hi = idx // 16
    lo = idx - hi * 16

    k3 = jax.lax.broadcasted_iota(jnp.int32, (ROWS, 16, CH), 1)
    big = jnp.zeros((ROWS * 16, ROWS * 16), jnp.float32)
    for c in range(NCHUNK):
        hi_c = hi[:, c * CH:(c + 1) * CH]
        lo_c = lo[:, c * CH:(c + 1) * CH]
        oh_hi = (hi_c[:, None, :] == k3).astype(jnp.float32).reshape(ROWS * 16, CH)
        oh_lo = (lo_c[:, None, :] == k3).astype(jnp.float32).reshape(ROWS * 16, CH)
        big = big + jax.lax.dot_general(
            oh_hi, oh_lo, (((1,), (1,)), ((), ())),
            preferred_element_type=jnp.float32)
    hist2d = jnp.zeros((16, 16), jnp.float32)
    for rr in range(ROWS):
        hist2d = hist2d + big[rr * 16:(rr + 1) * 16, rr * 16:(rr + 1) * 16]

    a_i = jax.lax.broadcasted_iota(jnp.int32, (16, 16), 0)
    b_j = jax.lax.broadcasted_iota(jnp.int32, (16, 16), 1)
    le16 = (a_i <= b_j).astype(jnp.float32)
    lt16 = (a_i < b_j).astype(jnp.float32)
    ones16 = jnp.ones((16, 16), jnp.float32)
    f = a_i * 16 + b_j
    centers = gmin + (f.astype(jnp.float32) + 0.5) / scale
    wc = hist2d * centers

    def flat_cumsum(m):
        pref = jax.lax.dot_general(m, le16, (((1,), (0,)), ((), ())),
                                   preferred_element_type=jnp.float32,
                                   precision=jax.lax.Precision.HIGHEST)
        rows = jax.lax.dot_general(m, ones16, (((1,), (0,)), ((), ())),
                                   preferred_element_type=jnp.float32,
                                   precision=jax.lax.Precision.HIGHEST)
        prev = jax.lax.dot_general(lt16, rows, (((0,), (0,)), ((), ())),
                                   preferred_element_type=jnp.float32,
                                   precision=jax.lax.Precision.HIGHEST)
        return pref + prev

    w1 = flat_cumsum(hist2d)
    c1 = flat_cumsum(wc)
    total_w = jnp.sum(hist2d)
    total_c = jnp.sum(wc)

    w2p = total_w - w1
    m1 = c1 / jnp.maximum(w1, 1e-12)
    m2 = (total_c - c1) / jnp.maximum(w2p, 1e-12)
    var12 = w1 * w2p * (m1 - m2) ** 2
    var12 = jnp.where(f < NBINS - 1, var12, -1.0)
    maxv = jnp.max(var12)
    arg = jnp.min(jnp.where(var12 == maxv, f, NBINS))
    thr = jnp.max(jnp.where(f == arg, centers, -jnp.inf))

    out_ref[0] = jnp.where(gray > thr, 255.0, 0.0)


def kernel(inputs):
    bsz = inputs.shape[0]
    wvec = jnp.array([0.2989, 0.587, 0.114], dtype=inputs.dtype)
    gray = jnp.tensordot(inputs, wvec, axes=[[-1], [0]])
    binary = pl.pallas_call(
        _otsu_body,
        grid=(bsz,),
        in_specs=[pl.BlockSpec((1, ROWS, COLS), lambda i: (i, 0, 0))],
        out_specs=pl.BlockSpec((1, ROWS, COLS), lambda i: (i, 0, 0)),
        out_shape=jax.ShapeDtypeStruct((bsz, ROWS, COLS), jnp.float32),
    )(gray.reshape(bsz, ROWS, COLS))
    binary = binary.reshape(bsz, H, W)
    return jnp.repeat(binary[..., None], 3, axis=-1)
